# baseline (speedup 1.0000x reference)
import functools

import jax
import jax.numpy as jnp
from jax import lax
from jax.experimental import pallas as pl
from jax.experimental.pallas import tpu as pltpu
from jax.experimental.pallas import tpu_sc as plsc

N = 10000
E = 160000
F = 256
HALF = 128
NPAD = 10240
NC = 2
NS = 16
CH = 128
C3 = 80
CP = 40
C1 = 40
EPAD = NS * C3 * CH
DUMMY = N
DEGW = 128
RB = 2000

RPT = NPAD // NS


def _deg_body(dst_hbm, ones_hbm, zrow_hbm, degp_hbm, deg_sp, dst_v, ones_v,
              ssem):
    c = lax.axis_index("c")
    s = lax.axis_index("s")
    w = c * NS + s
    pltpu.sync_copy(dst_hbm.at[w], dst_v)
    pltpu.sync_copy(ones_hbm, ones_v)
    pltpu.sync_copy(zrow_hbm, deg_sp.at[pl.ds(s * RPT, RPT)])
    plsc.subcore_barrier()

    def fire(j, carry):
        pltpu.async_copy(ones_v, deg_sp.at[dst_v.at[j]], ssem, add=True)
        return carry

    lax.fori_loop(0, C1, fire, 0, unroll=False)

    def drain(j, carry):
        pltpu.make_async_copy(ones_v, deg_sp.at[dst_v.at[j]], ssem).wait()
        return carry

    lax.fori_loop(0, C1, drain, 0, unroll=False)
    plsc.subcore_barrier()
    pltpu.sync_copy(
        deg_sp.at[pl.ds(s * RPT, RPT)],
        degp_hbm.at[pl.ds(c * NPAD + s * RPT, RPT)],
    )


def _agg_body(srcb_hbm, dstb_hbm, y_hbm, zeros_hbm, out_hbm,
              agg_sp, src_v, dst_v, bufs, gsems):
    c = lax.axis_index("c")
    s = lax.axis_index("s")
    w = c * NS + s
    pltpu.sync_copy(zeros_hbm, agg_sp.at[pl.ds(s * RPT, RPT)])
    plsc.subcore_barrier()

    NQ = 4
    QW = CH // NQ

    def gather(j, b):
        for q in range(NQ):
            pltpu.async_copy(
                y_hbm.at[src_v.at[j, pl.ds(q * QW, QW)]],
                bufs.at[b, pl.ds(q * QW, QW)],
                gsems.at[b],
            )

    def gather_wait(j, b):
        for q in range(NQ):
            pltpu.make_async_copy(
                y_hbm.at[src_v.at[j, pl.ds(q * QW, QW)]],
                bufs.at[b, pl.ds(q * QW, QW)],
                gsems.at[b],
            ).wait()

    def scat(j, b):
        pltpu.sync_copy(bufs.at[b], agg_sp.at[dst_v.at[j]], add=True)

    for p in range(2):
        pltpu.sync_copy(srcb_hbm.at[w, pl.ds(p * CP, CP)], src_v)
        pltpu.sync_copy(dstb_hbm.at[s, pl.ds(p * CP, CP)], dst_v)
        gather(0, 0)
        gather(1, 1)

        def step(i, carry):
            for b in range(2):
                j = i * 2 + b
                gather_wait(j, b)
                scat(j, b)

                @pl.when(j + 2 < CP)
                def _():
                    gather(j + 2, b)
            return carry

        lax.fori_loop(0, CP // 2, step, 0, unroll=False)
    plsc.subcore_barrier()
    pltpu.sync_copy(
        agg_sp.at[pl.ds(s * RPT, RPT)],
        out_hbm.at[pl.ds(c * NPAD + s * RPT, RPT)],
    )


@functools.lru_cache(maxsize=None)
def _sc_kernels():
    mesh = plsc.VectorSubcoreMesh(
        core_axis_name="c", subcore_axis_name="s", num_cores=NC, num_subcores=NS
    )
    deg_kernel = pl.kernel(
        _deg_body,
        out_type=jax.ShapeDtypeStruct((NC * NPAD, DEGW), jnp.float32),
        mesh=mesh,
        scratch_types=[
            pltpu.VMEM_SHARED((NPAD, DEGW), jnp.float32),
            pltpu.VMEM((C1, CH), jnp.int32),
            pltpu.VMEM((CH, DEGW), jnp.float32),
            pltpu.SemaphoreType.DMA,
        ],
    )
    agg_kernel = pl.kernel(
        _agg_body,
        out_type=jax.ShapeDtypeStruct((NC * NPAD, HALF), jnp.float32),
        mesh=mesh,
        scratch_types=[
            pltpu.VMEM_SHARED((NPAD, HALF), jnp.float32),
            pltpu.VMEM((CP, CH), jnp.int32),
            pltpu.VMEM((CP, CH), jnp.int32),
            pltpu.VMEM((2, CH, HALF), jnp.float32),
            pltpu.SemaphoreType.DMA((2,)),
        ],
    )
    return deg_kernel, agg_kernel


def _mm_body(x_ref, wc_ref, wres_ref, xw_ref, xres_ref):
    xw_ref[0] = lax.dot_general(
        x_ref[...], wc_ref[...], (((1,), (1,)), ((), ())),
        preferred_element_type=jnp.float32,
    )
    xres_ref[...] = lax.dot_general(
        x_ref[...], wres_ref[...], (((1,), (1,)), ((), ())),
        preferred_element_type=jnp.float32,
    )


def _scale_body(xw_ref, dega_ref, degb_ref, y_ref):
    deg = dega_ref[0, :, 0:1] + degb_ref[0, :, 0:1] + 1.0
    dinv = lax.rsqrt(jnp.maximum(deg, 1e-12))
    y_ref[0] = dinv * xw_ref[0]


def _final_body(xres_ref, b_ref, agg_ref, y_ref, dega_ref, degb_ref, out_ref):
    deg = dega_ref[0, :, 0:1] + degb_ref[0, :, 0:1] + 1.0
    dinv = lax.rsqrt(jnp.maximum(deg, 1e-12))
    g = dinv * (agg_ref[0] + y_ref[0]) + b_ref[...][None, :]
    out_ref[...] = jnp.maximum(g, 0.0) + xres_ref[...]


def kernel(x, edge_index, W_conv, b_conv, W_res):
    src = edge_index[0].astype(jnp.int32)
    dst = edge_index[1].astype(jnp.int32)
    n_pad_edges = EPAD - E
    src_p = jnp.concatenate([src, jnp.zeros((n_pad_edges,), jnp.int32)])
    dst_p = jnp.concatenate([dst, jnp.full((n_pad_edges,), DUMMY, jnp.int32)])

    dst1 = dst_p.reshape(NC * NS, C1, CH)
    s3 = src_p.reshape(NS, C3, CH)
    srcb = jnp.concatenate([s3, s3 + NPAD], axis=0)
    dstb = dst_p.reshape(NS, C3, CH)

    ones_d = jnp.ones((CH, DEGW), jnp.float32)
    zrow_d = jnp.zeros((RPT, DEGW), jnp.float32)

    deg_kernel, agg_kernel = _sc_kernels()
    degp = deg_kernel(dst1, ones_d, zrow_d)

    nrb = N // RB
    grid = (nrb, NC)
    xw3, xres = pl.pallas_call(
        _mm_body,
        grid=grid,
        in_specs=[
            pl.BlockSpec((RB, F), lambda i, h: (i, 0)),
            pl.BlockSpec((HALF, F), lambda i, h: (h, 0)),
            pl.BlockSpec((HALF, F), lambda i, h: (h, 0)),
        ],
        out_specs=[
            pl.BlockSpec((1, RB, HALF), lambda i, h: (h, i, 0)),
            pl.BlockSpec((RB, HALF), lambda i, h: (i, h)),
        ],
        out_shape=[
            jax.ShapeDtypeStruct((NC, NPAD, HALF), jnp.float32),
            jax.ShapeDtypeStruct((N, F), jnp.float32),
        ],
    )(x, W_conv, W_res)

    degp3 = degp.reshape(NC, NPAD, DEGW)[:, :, :8]
    y3 = pl.pallas_call(
        _scale_body,
        grid=grid,
        in_specs=[
            pl.BlockSpec((1, RB, HALF), lambda i, h: (h, i, 0)),
            pl.BlockSpec((1, RB, 8), lambda i, h: (0, i, 0)),
            pl.BlockSpec((1, RB, 8), lambda i, h: (1, i, 0)),
        ],
        out_specs=pl.BlockSpec((1, RB, HALF), lambda i, h: (h, i, 0)),
        out_shape=jax.ShapeDtypeStruct((NC, NPAD, HALF), jnp.float32),
    )(xw3, degp3, degp3)

    agg = agg_kernel(srcb, dstb, y3.reshape(NC * NPAD, HALF), zrow_d)

    out = pl.pallas_call(
        _final_body,
        grid=grid,
        in_specs=[
            pl.BlockSpec((RB, HALF), lambda i, h: (i, h)),
            pl.BlockSpec((HALF,), lambda i, h: (h,)),
            pl.BlockSpec((1, RB, HALF), lambda i, h: (h, i, 0)),
            pl.BlockSpec((1, RB, HALF), lambda i, h: (h, i, 0)),
            pl.BlockSpec((1, RB, 8), lambda i, h: (0, i, 0)),
            pl.BlockSpec((1, RB, 8), lambda i, h: (1, i, 0)),
        ],
        out_specs=pl.BlockSpec((RB, HALF), lambda i, h: (i, h)),
        out_shape=jax.ShapeDtypeStruct((N, F), jnp.float32),
    )(xres, b_conv, agg.reshape(NC, NPAD, HALF), y3, degp3, degp3)

    return out


# speedup vs baseline: 1.0052x; 1.0052x over previous
import functools

import jax
import jax.numpy as jnp
from jax import lax
from jax.experimental import pallas as pl
from jax.experimental.pallas import tpu as pltpu
from jax.experimental.pallas import tpu_sc as plsc

N = 10000
E = 160000
F = 256
HALF = 128
NPAD = 10240
NC = 2
NS = 16
CH = 128
C3 = 80
CP = 40
C1 = 40
EPAD = NS * C3 * CH
DUMMY = N
DEGW = 128
RB = 5000

RPT = NPAD // NS


def _deg_body(dst_hbm, ones_hbm, zrow_hbm, degp_hbm, deg_sp, dst_v, ones_v,
              ssem):
    c = lax.axis_index("c")
    s = lax.axis_index("s")
    w = c * NS + s
    pltpu.sync_copy(dst_hbm.at[w], dst_v)
    pltpu.sync_copy(ones_hbm, ones_v)
    pltpu.sync_copy(zrow_hbm, deg_sp.at[pl.ds(s * RPT, RPT)])
    plsc.subcore_barrier()

    def fire(j, carry):
        pltpu.async_copy(ones_v, deg_sp.at[dst_v.at[j]], ssem, add=True)
        return carry

    lax.fori_loop(0, C1, fire, 0, unroll=False)

    def drain(j, carry):
        pltpu.make_async_copy(ones_v, deg_sp.at[dst_v.at[j]], ssem).wait()
        return carry

    lax.fori_loop(0, C1, drain, 0, unroll=False)
    plsc.subcore_barrier()
    pltpu.sync_copy(
        deg_sp.at[pl.ds(s * RPT, RPT)],
        degp_hbm.at[pl.ds(c * NPAD + s * RPT, RPT)],
    )


def _agg_body(srcb_hbm, dstb_hbm, y_hbm, zeros_hbm, out_hbm,
              agg_sp, src_v, dst_v, bufs, gsems):
    c = lax.axis_index("c")
    s = lax.axis_index("s")
    w = c * NS + s
    pltpu.sync_copy(zeros_hbm, agg_sp.at[pl.ds(s * RPT, RPT)])
    plsc.subcore_barrier()

    NQ = 4
    QW = CH // NQ

    def gather(j, b):
        for q in range(NQ):
            pltpu.async_copy(
                y_hbm.at[src_v.at[j, pl.ds(q * QW, QW)]],
                bufs.at[b, pl.ds(q * QW, QW)],
                gsems.at[b],
            )

    def gather_wait(j, b):
        for q in range(NQ):
            pltpu.make_async_copy(
                y_hbm.at[src_v.at[j, pl.ds(q * QW, QW)]],
                bufs.at[b, pl.ds(q * QW, QW)],
                gsems.at[b],
            ).wait()

    def scat(j, b):
        pltpu.sync_copy(bufs.at[b], agg_sp.at[dst_v.at[j]], add=True)

    for p in range(2):
        pltpu.sync_copy(srcb_hbm.at[w, pl.ds(p * CP, CP)], src_v)
        pltpu.sync_copy(dstb_hbm.at[s, pl.ds(p * CP, CP)], dst_v)
        gather(0, 0)
        gather(1, 1)

        def step(i, carry):
            for b in range(2):
                j = i * 2 + b
                gather_wait(j, b)
                scat(j, b)

                @pl.when(j + 2 < CP)
                def _():
                    gather(j + 2, b)
            return carry

        lax.fori_loop(0, CP // 2, step, 0, unroll=False)
    plsc.subcore_barrier()
    pltpu.sync_copy(
        agg_sp.at[pl.ds(s * RPT, RPT)],
        out_hbm.at[pl.ds(c * NPAD + s * RPT, RPT)],
    )


@functools.lru_cache(maxsize=None)
def _sc_kernels():
    mesh = plsc.VectorSubcoreMesh(
        core_axis_name="c", subcore_axis_name="s", num_cores=NC, num_subcores=NS
    )
    deg_kernel = pl.kernel(
        _deg_body,
        out_type=jax.ShapeDtypeStruct((NC * NPAD, DEGW), jnp.float32),
        mesh=mesh,
        scratch_types=[
            pltpu.VMEM_SHARED((NPAD, DEGW), jnp.float32),
            pltpu.VMEM((C1, CH), jnp.int32),
            pltpu.VMEM((CH, DEGW), jnp.float32),
            pltpu.SemaphoreType.DMA,
        ],
    )
    agg_kernel = pl.kernel(
        _agg_body,
        out_type=jax.ShapeDtypeStruct((NC * NPAD, HALF), jnp.float32),
        mesh=mesh,
        scratch_types=[
            pltpu.VMEM_SHARED((NPAD, HALF), jnp.float32),
            pltpu.VMEM((CP, CH), jnp.int32),
            pltpu.VMEM((CP, CH), jnp.int32),
            pltpu.VMEM((2, CH, HALF), jnp.float32),
            pltpu.SemaphoreType.DMA((2,)),
        ],
    )
    return deg_kernel, agg_kernel


def _mm_body(x_ref, wc_ref, wres_ref, xw_ref, xres_ref):
    xw_ref[0] = lax.dot_general(
        x_ref[...], wc_ref[...], (((1,), (1,)), ((), ())),
        preferred_element_type=jnp.float32,
    )
    xres_ref[...] = lax.dot_general(
        x_ref[...], wres_ref[...], (((1,), (1,)), ((), ())),
        preferred_element_type=jnp.float32,
    )


def _scale_body(xw_ref, dega_ref, degb_ref, y_ref):
    deg = dega_ref[0, :, 0:1] + degb_ref[0, :, 0:1] + 1.0
    dinv = lax.rsqrt(jnp.maximum(deg, 1e-12))
    y_ref[0] = dinv * xw_ref[0]


def _final_body(xres_ref, b_ref, agg_ref, y_ref, dega_ref, degb_ref, out_ref):
    deg = dega_ref[0, :, 0:1] + degb_ref[0, :, 0:1] + 1.0
    dinv = lax.rsqrt(jnp.maximum(deg, 1e-12))
    g = dinv * (agg_ref[0] + y_ref[0]) + b_ref[...][None, :]
    out_ref[...] = jnp.maximum(g, 0.0) + xres_ref[...]


def kernel(x, edge_index, W_conv, b_conv, W_res):
    src = edge_index[0].astype(jnp.int32)
    dst = edge_index[1].astype(jnp.int32)
    n_pad_edges = EPAD - E
    src_p = jnp.concatenate([src, jnp.zeros((n_pad_edges,), jnp.int32)])
    dst_p = jnp.concatenate([dst, jnp.full((n_pad_edges,), DUMMY, jnp.int32)])

    dst1 = dst_p.reshape(NC * NS, C1, CH)
    s3 = src_p.reshape(NS, C3, CH)
    srcb = jnp.concatenate([s3, s3 + NPAD], axis=0)
    dstb = dst_p.reshape(NS, C3, CH)

    ones_d = jnp.ones((CH, DEGW), jnp.float32)
    zrow_d = jnp.zeros((RPT, DEGW), jnp.float32)

    deg_kernel, agg_kernel = _sc_kernels()
    degp = deg_kernel(dst1, ones_d, zrow_d)

    nrb = N // RB
    grid = (nrb, NC)
    xw3, xres = pl.pallas_call(
        _mm_body,
        grid=grid,
        in_specs=[
            pl.BlockSpec((RB, F), lambda i, h: (i, 0)),
            pl.BlockSpec((HALF, F), lambda i, h: (h, 0)),
            pl.BlockSpec((HALF, F), lambda i, h: (h, 0)),
        ],
        out_specs=[
            pl.BlockSpec((1, RB, HALF), lambda i, h: (h, i, 0)),
            pl.BlockSpec((RB, HALF), lambda i, h: (i, h)),
        ],
        out_shape=[
            jax.ShapeDtypeStruct((NC, NPAD, HALF), jnp.float32),
            jax.ShapeDtypeStruct((N, F), jnp.float32),
        ],
    )(x, W_conv, W_res)

    degp3 = degp.reshape(NC, NPAD, DEGW)
    y3 = pl.pallas_call(
        _scale_body,
        grid=grid,
        in_specs=[
            pl.BlockSpec((1, RB, HALF), lambda i, h: (h, i, 0)),
            pl.BlockSpec((1, RB, DEGW), lambda i, h: (0, i, 0)),
            pl.BlockSpec((1, RB, DEGW), lambda i, h: (1, i, 0)),
        ],
        out_specs=pl.BlockSpec((1, RB, HALF), lambda i, h: (h, i, 0)),
        out_shape=jax.ShapeDtypeStruct((NC, NPAD, HALF), jnp.float32),
    )(xw3, degp3, degp3)

    agg = agg_kernel(srcb, dstb, y3.reshape(NC * NPAD, HALF), zrow_d)

    out = pl.pallas_call(
        _final_body,
        grid=grid,
        in_specs=[
            pl.BlockSpec((RB, HALF), lambda i, h: (i, h)),
            pl.BlockSpec((HALF,), lambda i, h: (h,)),
            pl.BlockSpec((1, RB, HALF), lambda i, h: (h, i, 0)),
            pl.BlockSpec((1, RB, HALF), lambda i, h: (h, i, 0)),
            pl.BlockSpec((1, RB, DEGW), lambda i, h: (0, i, 0)),
            pl.BlockSpec((1, RB, DEGW), lambda i, h: (1, i, 0)),
        ],
        out_specs=pl.BlockSpec((RB, HALF), lambda i, h: (i, h)),
        out_shape=jax.ShapeDtypeStruct((N, F), jnp.float32),
    )(xres, b_conv, agg.reshape(NC, NPAD, HALF), y3, degp3, degp3)

    return out
